# baseline (speedup 1.0000x reference)
import jax
import jax.numpy as jnp
from jax.experimental import pallas as pl


def _combine(f_ref, d_ref, o_ref):
    o_ref[...] = f_ref[...] + 0.0 * d_ref[0, 0]


def kernel(history_data, future_data, batch_seen, epoch, train, dummy_param):
    b, w, f, c = future_data.shape
    flat = future_data.reshape(b * w, f * c)
    d2 = dummy_param.reshape(1, 1)
    out = pl.pallas_call(
        _combine,
        out_shape=jax.ShapeDtypeStruct((b * w, f * c), jnp.float32),
    )(flat, d2)
    return out.reshape(future_data.shape)
